# baseline (speedup 1.0000x reference)
import functools

import jax
import jax.numpy as jnp
from jax import lax
from jax.experimental import pallas as pl
from jax.experimental.pallas import tpu as pltpu
from jax.experimental.pallas import tpu_sc as plsc

DIM = 64
HB = 128
GH = 2


@functools.lru_cache(maxsize=None)
def _make_gather(B: int, H: int):
    info = plsc.get_sparse_core_info()
    nc = info.num_cores
    nw = nc * info.num_subcores
    nblk = B // HB
    blk_per_w = nblk // nw
    idx_per_w = blk_per_w * HB * H
    steps_per_blk = H // GH
    nsteps = blk_per_w * steps_per_blk
    rows_per_step = GH * HB
    assert B % (HB * nw) == 0 and H % GH == 0
    mesh = plsc.VectorSubcoreMesh(core_axis_name="c", subcore_axis_name="s")

    @functools.partial(
        pl.kernel,
        mesh=mesh,
        out_type=jax.ShapeDtypeStruct((H, DIM // 8, B // HB, 8, HB), jnp.float32),
        scratch_types=[
            pltpu.VMEM((idx_per_w,), jnp.int32),
            pltpu.VMEM((idx_per_w,), jnp.int32),
            pltpu.VMEM((2, rows_per_step, DIM), jnp.float32),
            pltpu.VMEM((2, GH, DIM // 8, 8, HB), jnp.float32),
            pltpu.SemaphoreType.DMA,
            pltpu.SemaphoreType.DMA,
        ],
        compiler_params=pltpu.CompilerParams(
            use_tc_tiling_on_sc=False, needs_layout_passes=False
        ),
    )
    def gather_kernel(idx_hbm, table_hbm, out_hbm, idx_v, idxT, gath, tile,
                      gsem, osem):
        wid = lax.axis_index("s") * nc + lax.axis_index("c")
        base = wid * idx_per_w
        iota = lax.iota(jnp.int32, 16)
        pltpu.sync_copy(idx_hbm.at[pl.ds(base, idx_per_w)], idx_v)

        def reorder(t, _):
            blk = t // H
            h = t % H
            for j in range(HB // 16):
                src = iota * H + (blk * HB * H + j * 16 * H + h)
                v = plsc.load_gather(idx_v, [src])
                idxT[pl.ds(blk * HB * H + h * HB + j * 16, 16)] = v
            return 0

        lax.fori_loop(0, blk_per_w * H, reorder, 0)

        def start_gather(g, p):
            pltpu.async_copy(
                table_hbm.at[idxT.at[pl.ds(g * rows_per_step, rows_per_step)]],
                gath.at[p],
                gsem,
            )

        def start_out(g, p):
            blk = g // steps_per_blk
            h0 = (g % steps_per_blk) * GH
            pltpu.async_copy(
                tile.at[p],
                out_hbm.at[pl.ds(h0, GH), :, wid * blk_per_w + blk],
                osem,
            )

        def wait_one(sem):
            pltpu.make_async_copy(gath.at[0], out_hbm.at[pl.ds(0, GH), :, 0],
                                  sem).wait()

        def transpose(p):
            def tbody(t, _):
                hh = t // (DIM // 8)
                c = t % (DIM // 8)
                for c8 in range(8):
                    col = jnp.full((16,), c * 8 + c8, jnp.int32)
                    for j in range(HB // 16):
                        row = iota + (hh * HB + j * 16)
                        v = plsc.load_gather(gath.at[p], [row, col])
                        tile[p, hh, c, c8, pl.ds(j * 16, 16)] = v
                return 0

            lax.fori_loop(0, GH * (DIM // 8), tbody, 0)

        start_gather(0, 0)

        def body(g2, _):
            for p in range(2):
                g = g2 * 2 + p
                pl.when(g >= 2)(lambda: wait_one(osem))
                pl.when(g + 1 < nsteps)(
                    functools.partial(start_gather_dyn, g + 1, 1 - p))
                wait_one(gsem)
                transpose(p)
                start_out(g, p)
            return 0

        def start_gather_dyn(g, p):
            start_gather(g, p)

        lax.fori_loop(0, nsteps // 2, body, 0)
        wait_one(osem)
        wait_one(osem)

    return gather_kernel


def kernel(input_, weight):
    b, h = input_.shape
    idx = input_.reshape(b * h).astype(jnp.int32)
    out5 = _make_gather(b, h)(idx, weight)
    return out5.transpose(2, 4, 0, 1, 3).reshape(b, h, DIM)
